# Initial kernel scaffold; baseline (speedup 1.0000x reference)
import jax
import jax.numpy as jnp
from jax.experimental import pallas as pl


def kernel(node_attrs, positions, edge_index, batch, ptr, shifts, cell, W_embed, W_rbf1, W_rbf2, W1, W2, W_readout):
    raise NotImplementedError("write your pallas kernel here")


# speedup vs baseline: 1.6778x; 1.6778x over previous
import functools

import numpy as np
import jax
import jax.numpy as jnp
from jax import lax
from jax.experimental import pallas as pl
from jax.experimental.pallas import tpu as pltpu
from jax.experimental.pallas import tpu_sc as plsc

N = 10000
E = 320000
HID = 128
NRBF = 8
NSYS = 8
RCUT = 5.0

NC = 2
NS = 16
NW = NC * NS
EPW = E // NW
C = 80
NCH = EPW // C
RPT = N // NS
DR = 125

_f32 = jnp.float32
_mesh = plsc.VectorSubcoreMesh(
    core_axis_name="c", subcore_axis_name="s", num_cores=NC, num_subcores=NS)


def _wid_base():
    cid = lax.axis_index("c")
    sid = lax.axis_index("s")
    return cid, sid, (sid * NC + cid) * EPW


def _zero_db(db, nrows):
    zero16 = jnp.zeros((16,), _f32)
    def body(i, _):
        for j in range(HID // 16):
            db[i, pl.ds(j * 16, 16)] = zero16
        return 0
    lax.fori_loop(0, nrows, body, 0)


def _drain(acc, db, out_hbm, cid, sid):
    for k in range(RPT // DR):
        r0 = sid * RPT + k * DR
        pltpu.sync_copy(acc.at[pl.ds(r0, DR)], db)
        pltpu.sync_copy(db, out_hbm.at[pl.ds(cid * N + r0, DR)])


def _zero_acc(acc, db, sid):
    _zero_db(db, DR)
    for k in range(RPT // DR):
        r0 = sid * RPT + k * DR
        pltpu.sync_copy(db, acc.at[pl.ds(r0, DR)])


@functools.partial(
    pl.kernel,
    out_type=(jax.ShapeDtypeStruct((E, 16), _f32),
              jax.ShapeDtypeStruct((E, 16), _f32)),
    mesh=_mesh,
    compiler_params=pltpu.CompilerParams(use_tc_tiling_on_sc=False),
    scratch_types=(pltpu.VMEM((C,), jnp.int32),
                   pltpu.VMEM((C,), jnp.int32),
                   pltpu.VMEM((C, 16), _f32),
                   pltpu.VMEM((C, 16), _f32),
                   pltpu.SemaphoreType.DMA),
)
def _sc_gather_pos(pos_hbm, src_hbm, dst_hbm, gs_hbm, gd_hbm,
                   idx_s, idx_d, bs, bd, sem):
    _, _, base = _wid_base()

    def body(i, _):
        st = base + i * C
        pltpu.sync_copy(src_hbm.at[pl.ds(st, C)], idx_s)
        pltpu.sync_copy(dst_hbm.at[pl.ds(st, C)], idx_d)
        pltpu.async_copy(pos_hbm.at[idx_s], bs, sem).wait()
        pltpu.async_copy(pos_hbm.at[idx_d], bd, sem).wait()
        pltpu.sync_copy(bs, gs_hbm.at[pl.ds(st, C)])
        pltpu.sync_copy(bd, gd_hbm.at[pl.ds(st, C)])
        return 0
    lax.fori_loop(0, NCH, body, 0)


@functools.partial(
    pl.kernel,
    out_type=jax.ShapeDtypeStruct((NC * N, HID), _f32),
    mesh=_mesh,
    compiler_params=pltpu.CompilerParams(use_tc_tiling_on_sc=False),
    scratch_types=(pltpu.VMEM((C,), jnp.int32),
                   pltpu.VMEM((C,), jnp.int32),
                   pltpu.VMEM((C, HID), _f32),
                   pltpu.VMEM((C, HID), _f32),
                   pltpu.VMEM((DR, HID), _f32),
                   pltpu.MemorySpace.VMEM_SHARED((N, HID), _f32),
                   pltpu.SemaphoreType.DMA),
)
def _sc_msg(h_hbm, ew_hbm, src_hbm, dst_hbm, out_hbm,
            idx_s, idx_d, rows, ewb, db, acc, sem):
    cid, sid, base = _wid_base()
    _zero_acc(acc, db, sid)
    plsc.subcore_barrier()

    def body(i, _):
        st = base + i * C
        pltpu.sync_copy(src_hbm.at[pl.ds(st, C)], idx_s)
        pltpu.sync_copy(dst_hbm.at[pl.ds(st, C)], idx_d)
        pltpu.async_copy(h_hbm.at[idx_s], rows, sem).wait()
        pltpu.sync_copy(ew_hbm.at[pl.ds(st, C)], ewb)

        def mul(e, _):
            for j in range(HID // 16):
                sl = pl.ds(j * 16, 16)
                rows[e, sl] = rows[e, sl] * ewb[e, sl]
            return 0
        lax.fori_loop(0, C, mul, 0)
        pltpu.sync_copy(rows, acc.at[idx_d], add=True)
        return 0
    lax.fori_loop(0, NCH, body, 0)
    plsc.subcore_barrier()
    _drain(acc, db, out_hbm, cid, sid)


@functools.partial(
    pl.kernel,
    out_type=(jax.ShapeDtypeStruct((E, HID), _f32),
              jax.ShapeDtypeStruct((NC * N, HID), _f32)),
    mesh=_mesh,
    compiler_params=pltpu.CompilerParams(use_tc_tiling_on_sc=False),
    scratch_types=(pltpu.VMEM((C,), jnp.int32),
                   pltpu.VMEM((C,), jnp.int32),
                   pltpu.VMEM((C, HID), _f32),
                   pltpu.VMEM((C, HID), _f32),
                   pltpu.VMEM((C, HID), _f32),
                   pltpu.VMEM((DR, HID), _f32),
                   pltpu.MemorySpace.VMEM_SHARED((N, HID), _f32),
                   pltpu.SemaphoreType.DMA),
)
def _sc_bwd2(g2_hbm, h1_hbm, ew_hbm, src_hbm, dst_hbm, prod_hbm, out_hbm,
             idx_s, idx_d, g2r, h1r, ewb, db, acc, sem):
    cid, sid, base = _wid_base()
    _zero_acc(acc, db, sid)
    plsc.subcore_barrier()

    def body(i, _):
        st = base + i * C
        pltpu.sync_copy(src_hbm.at[pl.ds(st, C)], idx_s)
        pltpu.sync_copy(dst_hbm.at[pl.ds(st, C)], idx_d)
        pltpu.async_copy(g2_hbm.at[idx_d], g2r, sem).wait()
        pltpu.async_copy(h1_hbm.at[idx_s], h1r, sem).wait()
        pltpu.sync_copy(ew_hbm.at[pl.ds(st, C)], ewb)

        def mul(e, _):
            for j in range(HID // 16):
                sl = pl.ds(j * 16, 16)
                g = g2r[e, sl]
                ewb[e, sl] = g * ewb[e, sl]
                h1r[e, sl] = g * h1r[e, sl]
            return 0
        lax.fori_loop(0, C, mul, 0)
        pltpu.sync_copy(h1r, prod_hbm.at[pl.ds(st, C)])
        pltpu.sync_copy(ewb, acc.at[idx_s], add=True)
        return 0
    lax.fori_loop(0, NCH, body, 0)
    plsc.subcore_barrier()
    _drain(acc, db, out_hbm, cid, sid)


@functools.partial(
    pl.kernel,
    out_type=jax.ShapeDtypeStruct((E, HID), _f32),
    mesh=_mesh,
    compiler_params=pltpu.CompilerParams(use_tc_tiling_on_sc=False),
    scratch_types=(pltpu.VMEM((C,), jnp.int32),
                   pltpu.VMEM((C,), jnp.int32),
                   pltpu.VMEM((C, HID), _f32),
                   pltpu.VMEM((C, HID), _f32),
                   pltpu.SemaphoreType.DMA),
)
def _sc_bwd1(g1_hbm, h0_hbm, src_hbm, dst_hbm, prod_hbm,
             idx_s, idx_d, g1r, h0r, sem):
    _, _, base = _wid_base()

    def body(i, _):
        st = base + i * C
        pltpu.sync_copy(src_hbm.at[pl.ds(st, C)], idx_s)
        pltpu.sync_copy(dst_hbm.at[pl.ds(st, C)], idx_d)
        pltpu.async_copy(g1_hbm.at[idx_d], g1r, sem).wait()
        pltpu.async_copy(h0_hbm.at[idx_s], h0r, sem).wait()

        def mul(e, _):
            for j in range(HID // 16):
                sl = pl.ds(j * 16, 16)
                h0r[e, sl] = g1r[e, sl] * h0r[e, sl]
            return 0
        lax.fori_loop(0, C, mul, 0)
        pltpu.sync_copy(h0r, prod_hbm.at[pl.ds(st, C)])
        return 0
    lax.fori_loop(0, NCH, body, 0)


@functools.partial(
    pl.kernel,
    out_type=jax.ShapeDtypeStruct((NC * N, 16), _f32),
    mesh=_mesh,
    compiler_params=pltpu.CompilerParams(use_tc_tiling_on_sc=False),
    scratch_types=(pltpu.VMEM((C,), jnp.int32),
                   pltpu.VMEM((C,), jnp.int32),
                   pltpu.VMEM((C, 16), _f32),
                   pltpu.VMEM((C, 16), _f32),
                   pltpu.VMEM((RPT, 16), _f32),
                   pltpu.MemorySpace.VMEM_SHARED((N, 16), _f32),
                   pltpu.SemaphoreType.DMA),
)
def _sc_forces(gv_hbm, src_hbm, dst_hbm, out_hbm,
               idx_s, idx_d, gvb, ngb, db, acc, sem):
    cid, sid, base = _wid_base()
    zero16 = jnp.zeros((16,), _f32)

    def zbody(i, _):
        db[i, :] = zero16
        return 0
    lax.fori_loop(0, RPT, zbody, 0)
    pltpu.sync_copy(db, acc.at[pl.ds(sid * RPT, RPT)])
    plsc.subcore_barrier()

    def body(i, _):
        st = base + i * C
        pltpu.sync_copy(src_hbm.at[pl.ds(st, C)], idx_s)
        pltpu.sync_copy(dst_hbm.at[pl.ds(st, C)], idx_d)
        pltpu.sync_copy(gv_hbm.at[pl.ds(st, C)], gvb)

        def neg(e, _):
            ngb[e, :] = -gvb[e, :]
            return 0
        lax.fori_loop(0, C, neg, 0)
        pltpu.sync_copy(gvb, acc.at[idx_d], add=True)
        pltpu.sync_copy(ngb, acc.at[idx_s], add=True)
        return 0
    lax.fori_loop(0, NCH, body, 0)
    plsc.subcore_barrier()
    pltpu.sync_copy(acc.at[pl.ds(sid * RPT, RPT)], db)
    pltpu.sync_copy(db, out_hbm.at[pl.ds(cid * N + sid * RPT, RPT)])


BE = 2000
BN = 2000
_mu = None


_MU = np.linspace(0.0, RCUT, NRBF, dtype=np.float32)


def _rbf_env(r, mu2d):
    mu = mu2d[0, :]
    rbf0 = jnp.exp(-2.0 * (r[:, None] - mu[None, :]) ** 2)
    u = r / RCUT
    env = jnp.where(u < 1.0, 0.5 * (jnp.cos(jnp.pi * u) + 1.0), 0.0)
    return mu, rbf0, u, env


def _tc_geom(mu2d, gs, gd, sh, wr1, wr2, vec4, ew1, ew2):
    v = gd[:, :4] - gs[:, :4] + sh[...]
    r = jnp.sqrt(jnp.sum(v * v, axis=1) + 1e-9)
    _, rbf0, _, env = _rbf_env(r, mu2d[...])
    rbf = rbf0 * env[:, None]
    vec4[...] = v
    ew1[...] = jnp.dot(rbf, wr1[...], preferred_element_type=_f32)
    ew2[...] = jnp.dot(rbf, wr2[...], preferred_element_type=_f32)


def _tc_embed(attrs, we, h0):
    h0[...] = jnp.dot(attrs[...], we[...], preferred_element_type=_f32)


def _silu_pair(a):
    sg = 1.0 / (1.0 + jnp.exp(-a))
    return a * sg, sg * (1.0 + a * (1.0 - sg))


def _tc_layer(h, m0, m1, w, h_out, s_out):
    a = jnp.dot(h[...] + m0[...] + m1[...], w[...],
                preferred_element_type=_f32)
    ho, so = _silu_pair(a)
    h_out[...] = ho
    s_out[...] = so


def _tc_head(h1, m0, m1, w2, w2t, wr, ne, g2):
    a = jnp.dot(h1[...] + m0[...] + m1[...], w2[...],
                preferred_element_type=_f32)
    h2, s2 = _silu_pair(a)
    wrow = wr[...]
    ne[...] = jnp.sum(h2 * wrow, axis=1)[:, None]
    g2[...] = jnp.dot(s2 * wrow, w2t[...], preferred_element_type=_f32)


def _tc_energy(ne2d, out):
    out[...] = jnp.sum(ne2d[...], axis=1)


def _tc_bwd_dense(g2, q0, q1, s1, w1t, g1):
    g1[...] = jnp.dot((g2[...] + q0[...] + q1[...]) * s1[...], w1t[...],
                      preferred_element_type=_f32)


def _tc_gvec(mu2d, p1, p2, w1t, w2t, vec4, gv16):
    grbf = (jnp.dot(p1[...], w1t[...], preferred_element_type=_f32)
            + jnp.dot(p2[...], w2t[...], preferred_element_type=_f32))
    v = vec4[...]
    r = jnp.sqrt(jnp.sum(v * v, axis=1) + 1e-9)
    mu, rbf0, u, env = _rbf_env(r, mu2d[...])
    denv = jnp.where(u < 1.0,
                     -0.5 * jnp.pi * jnp.sin(jnp.pi * u) / RCUT, 0.0)
    drbf0 = -4.0 * (r[:, None] - mu[None, :]) * rbf0
    g_r = jnp.sum(grbf * (drbf0 * env[:, None] + rbf0 * denv[:, None]),
                  axis=1)
    gv = (g_r / r)[:, None] * v
    gv16[...] = jnp.concatenate(
        [gv, jnp.zeros((gv.shape[0], 12), _f32)], axis=1)


def _full(shape):
    zeros = (0,) * len(shape)
    return pl.BlockSpec(shape, lambda *_: zeros)


def _erow(width):
    return pl.BlockSpec((BE, width), lambda i: (i, 0))


def _nrow(width):
    return pl.BlockSpec((BN, width), lambda i: (i, 0))


def kernel(node_attrs, positions, edge_index, batch, ptr, shifts, cell,
           W_embed, W_rbf1, W_rbf2, W1, W2, W_readout):
    f32 = _f32
    src = edge_index[0].astype(jnp.int32)
    dst = edge_index[1].astype(jnp.int32)
    pos16 = jnp.zeros((N, 16), f32).at[:, :3].set(positions)
    sh4 = jnp.zeros((E, 4), f32).at[:, :3].set(shifts)
    mu2d = jnp.asarray(_MU).reshape(1, NRBF)

    gs, gd = _sc_gather_pos(pos16, src, dst)

    vec4, ew1, ew2 = pl.pallas_call(
        _tc_geom,
        grid=(E // BE,),
        in_specs=[_full((1, NRBF)), _erow(16), _erow(16), _erow(4),
                  _full((NRBF, HID)), _full((NRBF, HID))],
        out_specs=[_erow(4), _erow(HID), _erow(HID)],
        out_shape=[jax.ShapeDtypeStruct((E, 4), f32),
                   jax.ShapeDtypeStruct((E, HID), f32),
                   jax.ShapeDtypeStruct((E, HID), f32)],
    )(mu2d, gs, gd, sh4, W_rbf1, W_rbf2)

    h0 = pl.pallas_call(
        _tc_embed,
        grid=(N // BN,),
        in_specs=[_nrow(node_attrs.shape[1]), _full(W_embed.shape)],
        out_specs=_nrow(HID),
        out_shape=jax.ShapeDtypeStruct((N, HID), f32),
    )(node_attrs, W_embed)

    m1p = _sc_msg(h0, ew1, src, dst)

    h1, s1 = pl.pallas_call(
        _tc_layer,
        grid=(N // BN,),
        in_specs=[_nrow(HID), _nrow(HID), _nrow(HID), _full((HID, HID))],
        out_specs=[_nrow(HID), _nrow(HID)],
        out_shape=[jax.ShapeDtypeStruct((N, HID), f32),
                   jax.ShapeDtypeStruct((N, HID), f32)],
    )(h0, m1p[:N], m1p[N:], W1)

    m2p = _sc_msg(h1, ew2, src, dst)

    wrow = W_readout.reshape(1, HID)
    node_e, G2 = pl.pallas_call(
        _tc_head,
        grid=(N // BN,),
        in_specs=[_nrow(HID), _nrow(HID), _nrow(HID),
                  _full((HID, HID)), _full((HID, HID)), _full((1, HID))],
        out_specs=[_nrow(1), _nrow(HID)],
        out_shape=[jax.ShapeDtypeStruct((N, 1), f32),
                   jax.ShapeDtypeStruct((N, HID), f32)],
    )(h1, m2p[:N], m2p[N:], W2, W2.T, wrow)

    energy = pl.pallas_call(
        _tc_energy,
        in_specs=[_full((NSYS, N // NSYS))],
        out_specs=_full((NSYS,)),
        out_shape=jax.ShapeDtypeStruct((NSYS,), f32),
    )(node_e.reshape(NSYS, N // NSYS))

    prod2, q = _sc_bwd2(G2, h1, ew2, src, dst)

    G1 = pl.pallas_call(
        _tc_bwd_dense,
        grid=(N // BN,),
        in_specs=[_nrow(HID), _nrow(HID), _nrow(HID), _nrow(HID),
                  _full((HID, HID))],
        out_specs=_nrow(HID),
        out_shape=jax.ShapeDtypeStruct((N, HID), f32),
    )(G2, q[:N], q[N:], s1, W1.T)

    prod1 = _sc_bwd1(G1, h0, src, dst)

    gv16 = pl.pallas_call(
        _tc_gvec,
        grid=(E // BE,),
        in_specs=[_full((1, NRBF)), _erow(HID), _erow(HID),
                  _full((HID, NRBF)), _full((HID, NRBF)), _erow(4)],
        out_specs=_erow(16),
        out_shape=jax.ShapeDtypeStruct((E, 16), f32),
    )(mu2d, prod1, prod2, W_rbf1.T, W_rbf2.T, vec4)

    fp = _sc_forces(gv16, src, dst)
    forces = -(fp[:N, :3] + fp[N:, :3])
    return energy, forces


# speedup vs baseline: 2.3703x; 1.4127x over previous
import functools

import numpy as np
import jax
import jax.numpy as jnp
from jax import lax
from jax.experimental import pallas as pl
from jax.experimental.pallas import tpu as pltpu
from jax.experimental.pallas import tpu_sc as plsc

N = 10000
E = 320000
HID = 128
NRBF = 8
NSYS = 8
RCUT = 5.0

NC = 2
NS = 16
NW = NC * NS
EPW = E // NW
C = 80
C4 = 40
NCH = EPW // C
RPT = N // NS
DR = 25

_f32 = jnp.float32
_mesh = plsc.VectorSubcoreMesh(
    core_axis_name="c", subcore_axis_name="s", num_cores=NC, num_subcores=NS)
_params = pltpu.CompilerParams(use_tc_tiling_on_sc=False)


def _wid_base():
    cid = lax.axis_index("c")
    sid = lax.axis_index("s")
    return cid, sid, (sid * NC + cid) * EPW


def _zero_db(db, nrows):
    zero16 = jnp.zeros((16,), _f32)

    def body(i, _):
        for j in range(HID // 16):
            db[i, pl.ds(j * 16, 16)] = zero16
        return 0
    lax.fori_loop(0, nrows, body, 0)


def _drain(acc, db, out_hbm, cid, sid):
    for k in range(RPT // DR):
        r0 = sid * RPT + k * DR
        pltpu.sync_copy(acc.at[pl.ds(r0, DR)], db)
        pltpu.sync_copy(db, out_hbm.at[pl.ds(cid * N + r0, DR)])


def _zero_acc(acc, db, sid):
    _zero_db(db, DR)
    for k in range(RPT // DR):
        r0 = sid * RPT + k * DR
        pltpu.sync_copy(db, acc.at[pl.ds(r0, DR)])


def _pipe(nch, issue_lin, wait_lin, issue_gath, wait_gath, work, wait_w):
    issue_lin(0, 0)
    issue_lin(1, 1)
    wait_lin(0)
    issue_gath(0, 0)

    def it(i, _):
        b = i % 2
        o = 1 - b

        @pl.when(jnp.logical_and(i + 1 < nch, i >= 1))
        def _dw():
            wait_w(o)

        @pl.when(i + 1 < nch)
        def _ig():
            wait_lin(o)
            issue_gath(i + 1, o)

        wait_gath(b)
        work(i, b)

        @pl.when(i + 2 < nch)
        def _il():
            issue_lin(i + 2, b)
        return 0
    lax.fori_loop(0, nch, it, 0)
    wait_w(0)
    wait_w(1)


def _mk_lin(base, sem, pairs, c=C):
    def issue(i, b):
        st = base + i * c
        for hbm, buf2 in pairs:
            pltpu.async_copy(hbm.at[pl.ds(st, c)], buf2.at[b], sem.at[b])

    def wait(b):
        for hbm, buf2 in pairs:
            pltpu.make_async_copy(hbm.at[pl.ds(0, c)], buf2.at[b],
                                  sem.at[b]).wait()
    return issue, wait


def _mk_gath(sem, trips):
    def issue(i, b):
        for tab, idx2, rows2 in trips:
            pltpu.async_copy(tab.at[idx2.at[b]], rows2.at[b], sem.at[b])

    def wait(b):
        for tab, idx2, rows2 in trips:
            pltpu.make_async_copy(tab.at[idx2.at[b]], rows2.at[b],
                                  sem.at[b]).wait()
    return issue, wait


def _noop(*_a):
    return None


@functools.partial(
    pl.kernel,
    out_type=(jax.ShapeDtypeStruct((E, 16), _f32),
              jax.ShapeDtypeStruct((E, 16), _f32)),
    mesh=_mesh,
    compiler_params=_params,
    scratch_types=(pltpu.VMEM((2, C), jnp.int32),
                   pltpu.VMEM((2, C), jnp.int32),
                   pltpu.VMEM((2, C, 16), _f32),
                   pltpu.VMEM((2, C, 16), _f32),
                   pltpu.SemaphoreType.DMA((2,)),
                   pltpu.SemaphoreType.DMA((2,)),
                   pltpu.SemaphoreType.DMA((2,))),
)
def _sc_gather_pos(pos_hbm, src_hbm, dst_hbm, gs_hbm, gd_hbm,
                   idx_s, idx_d, bs, bd, sem_l, sem_g, sem_w):
    _, _, base = _wid_base()
    issue_lin, wait_lin = _mk_lin(base, sem_l,
                                  [(src_hbm, idx_s), (dst_hbm, idx_d)])
    issue_g, wait_g = _mk_gath(sem_g, [(pos_hbm, idx_s, bs),
                                       (pos_hbm, idx_d, bd)])

    def work(i, b):
        st = base + i * C
        pltpu.async_copy(bs.at[b], gs_hbm.at[pl.ds(st, C)], sem_w.at[b])
        pltpu.async_copy(bd.at[b], gd_hbm.at[pl.ds(st, C)], sem_w.at[b])

    def wait_w(b):
        pltpu.make_async_copy(bs.at[b], gs_hbm.at[pl.ds(0, C)],
                              sem_w.at[b]).wait()
        pltpu.make_async_copy(bd.at[b], gd_hbm.at[pl.ds(0, C)],
                              sem_w.at[b]).wait()

    _pipe(NCH, issue_lin, wait_lin, issue_g, wait_g, work, wait_w)


@functools.partial(
    pl.kernel,
    out_type=jax.ShapeDtypeStruct((NC * N, HID), _f32),
    mesh=_mesh,
    compiler_params=_params,
    scratch_types=(pltpu.VMEM((2, C), jnp.int32),
                   pltpu.VMEM((2, C), jnp.int32),
                   pltpu.VMEM((2, C, HID), _f32),
                   pltpu.VMEM((2, C, HID), _f32),
                   pltpu.VMEM((DR, HID), _f32),
                   pltpu.MemorySpace.VMEM_SHARED((N, HID), _f32),
                   pltpu.SemaphoreType.DMA((2,)),
                   pltpu.SemaphoreType.DMA((2,))),
)
def _sc_msg(h_hbm, ew_hbm, src_hbm, dst_hbm, out_hbm,
            idx_s, idx_d, rows, ewb, db, acc, sem_l, sem_g):
    cid, sid, base = _wid_base()
    _zero_acc(acc, db, sid)
    plsc.subcore_barrier()

    issue_lin, wait_lin = _mk_lin(
        base, sem_l,
        [(src_hbm, idx_s), (dst_hbm, idx_d), (ew_hbm, ewb)])
    issue_g, wait_g = _mk_gath(sem_g, [(h_hbm, idx_s, rows)])

    def work(i, b):
        rb = rows.at[b]
        eb = ewb.at[b]

        @plsc.parallel_loop(0, C, 1, unroll=4)
        def _m(e):
            for j in range(HID // 16):
                sl = pl.ds(j * 16, 16)
                rb[e, sl] = rb[e, sl] * eb[e, sl]
        pltpu.sync_copy(rows.at[b], acc.at[idx_d.at[b]], add=True)

    _pipe(NCH, issue_lin, wait_lin, issue_g, wait_g, work, _noop)
    plsc.subcore_barrier()
    _drain(acc, db, out_hbm, cid, sid)


@functools.partial(
    pl.kernel,
    out_type=(jax.ShapeDtypeStruct((E, HID), _f32),
              jax.ShapeDtypeStruct((NC * N, HID), _f32)),
    mesh=_mesh,
    compiler_params=_params,
    scratch_types=(pltpu.VMEM((2, C4), jnp.int32),
                   pltpu.VMEM((2, C4), jnp.int32),
                   pltpu.VMEM((2, C4, HID), _f32),
                   pltpu.VMEM((2, C4, HID), _f32),
                   pltpu.VMEM((2, C4, HID), _f32),
                   pltpu.VMEM((DR, HID), _f32),
                   pltpu.MemorySpace.VMEM_SHARED((N, HID), _f32),
                   pltpu.SemaphoreType.DMA((2,)),
                   pltpu.SemaphoreType.DMA((2,)),
                   pltpu.SemaphoreType.DMA((2,))),
)
def _sc_bwd2(g2_hbm, h1_hbm, ew_hbm, src_hbm, dst_hbm, prod_hbm, out_hbm,
             idx_s, idx_d, g2r, h1r, ewb, db, acc, sem_l, sem_g, sem_w):
    cid, sid, base = _wid_base()
    _zero_acc(acc, db, sid)
    plsc.subcore_barrier()

    issue_lin, wait_lin = _mk_lin(
        base, sem_l,
        [(src_hbm, idx_s), (dst_hbm, idx_d), (ew_hbm, ewb)], c=C4)
    issue_g, wait_g = _mk_gath(sem_g, [(g2_hbm, idx_d, g2r),
                                       (h1_hbm, idx_s, h1r)])

    def work(i, b):
        gb = g2r.at[b]
        hb = h1r.at[b]
        eb = ewb.at[b]

        @plsc.parallel_loop(0, C4, 1, unroll=4)
        def _m(e):
            for j in range(HID // 16):
                sl = pl.ds(j * 16, 16)
                g = gb[e, sl]
                eb[e, sl] = g * eb[e, sl]
                hb[e, sl] = g * hb[e, sl]
        st = base + i * C4
        pltpu.async_copy(h1r.at[b], prod_hbm.at[pl.ds(st, C4)], sem_w.at[b])
        pltpu.sync_copy(ewb.at[b], acc.at[idx_s.at[b]], add=True)

    def wait_w(b):
        pltpu.make_async_copy(h1r.at[b], prod_hbm.at[pl.ds(0, C4)],
                              sem_w.at[b]).wait()

    _pipe(EPW // C4, issue_lin, wait_lin, issue_g, wait_g, work, wait_w)
    plsc.subcore_barrier()
    _drain(acc, db, out_hbm, cid, sid)


@functools.partial(
    pl.kernel,
    out_type=jax.ShapeDtypeStruct((E, HID), _f32),
    mesh=_mesh,
    compiler_params=_params,
    scratch_types=(pltpu.VMEM((2, C), jnp.int32),
                   pltpu.VMEM((2, C), jnp.int32),
                   pltpu.VMEM((2, C, HID), _f32),
                   pltpu.VMEM((2, C, HID), _f32),
                   pltpu.SemaphoreType.DMA((2,)),
                   pltpu.SemaphoreType.DMA((2,)),
                   pltpu.SemaphoreType.DMA((2,))),
)
def _sc_bwd1(g1_hbm, h0_hbm, src_hbm, dst_hbm, prod_hbm,
             idx_s, idx_d, g1r, h0r, sem_l, sem_g, sem_w):
    _, _, base = _wid_base()
    issue_lin, wait_lin = _mk_lin(base, sem_l,
                                  [(src_hbm, idx_s), (dst_hbm, idx_d)])
    issue_g, wait_g = _mk_gath(sem_g, [(g1_hbm, idx_d, g1r),
                                       (h0_hbm, idx_s, h0r)])

    def work(i, b):
        gb = g1r.at[b]
        hb = h0r.at[b]

        @plsc.parallel_loop(0, C, 1, unroll=4)
        def _m(e):
            for j in range(HID // 16):
                sl = pl.ds(j * 16, 16)
                hb[e, sl] = gb[e, sl] * hb[e, sl]
        st = base + i * C
        pltpu.async_copy(h0r.at[b], prod_hbm.at[pl.ds(st, C)], sem_w.at[b])

    def wait_w(b):
        pltpu.make_async_copy(h0r.at[b], prod_hbm.at[pl.ds(0, C)],
                              sem_w.at[b]).wait()

    _pipe(NCH, issue_lin, wait_lin, issue_g, wait_g, work, wait_w)


@functools.partial(
    pl.kernel,
    out_type=jax.ShapeDtypeStruct((NC * N, 16), _f32),
    mesh=_mesh,
    compiler_params=_params,
    scratch_types=(pltpu.VMEM((2, C), jnp.int32),
                   pltpu.VMEM((2, C), jnp.int32),
                   pltpu.VMEM((2, C, 16), _f32),
                   pltpu.VMEM((2, C, 16), _f32),
                   pltpu.VMEM((RPT, 16), _f32),
                   pltpu.MemorySpace.VMEM_SHARED((N, 16), _f32),
                   pltpu.SemaphoreType.DMA((2,))),
)
def _sc_forces(gv_hbm, src_hbm, dst_hbm, out_hbm,
               idx_s, idx_d, gvb, ngb, db, acc, sem_l):
    cid, sid, base = _wid_base()
    zero16 = jnp.zeros((16,), _f32)

    def zbody(i, _):
        db[i, :] = zero16
        return 0
    lax.fori_loop(0, RPT, zbody, 0)
    pltpu.sync_copy(db, acc.at[pl.ds(sid * RPT, RPT)])
    plsc.subcore_barrier()

    issue_lin, wait_lin = _mk_lin(
        base, sem_l,
        [(src_hbm, idx_s), (dst_hbm, idx_d), (gv_hbm, gvb)])

    def work(i, b):
        gb = gvb.at[b]
        nb = ngb.at[b]

        @plsc.parallel_loop(0, C, 1, unroll=4)
        def _m(e):
            nb[e, :] = -gb[e, :]
        pltpu.sync_copy(gvb.at[b], acc.at[idx_d.at[b]], add=True)
        pltpu.sync_copy(ngb.at[b], acc.at[idx_s.at[b]], add=True)

    _pipe(NCH, issue_lin, wait_lin, _noop, _noop, work, _noop)
    plsc.subcore_barrier()
    pltpu.sync_copy(acc.at[pl.ds(sid * RPT, RPT)], db)
    pltpu.sync_copy(db, out_hbm.at[pl.ds(cid * N + sid * RPT, RPT)])


BE = 2000
BN = 2000

_MU = np.linspace(0.0, RCUT, NRBF, dtype=np.float32)


def _rbf_env(r, mu2d):
    mu = mu2d[0, :]
    rbf0 = jnp.exp(-2.0 * (r[:, None] - mu[None, :]) ** 2)
    u = r / RCUT
    env = jnp.where(u < 1.0, 0.5 * (jnp.cos(jnp.pi * u) + 1.0), 0.0)
    return mu, rbf0, u, env


def _tc_geom(mu2d, gs, gd, sh, wr1, wr2, vec4, ew1, ew2):
    v = gd[:, :4] - gs[:, :4] + sh[...]
    r = jnp.sqrt(jnp.sum(v * v, axis=1) + 1e-9)
    _, rbf0, _, env = _rbf_env(r, mu2d[...])
    rbf = rbf0 * env[:, None]
    vec4[...] = v
    ew1[...] = jnp.dot(rbf, wr1[...], preferred_element_type=_f32)
    ew2[...] = jnp.dot(rbf, wr2[...], preferred_element_type=_f32)


def _tc_embed(attrs, we, h0):
    h0[...] = jnp.dot(attrs[...], we[...], preferred_element_type=_f32)


def _silu_pair(a):
    sg = 1.0 / (1.0 + jnp.exp(-a))
    return a * sg, sg * (1.0 + a * (1.0 - sg))


def _tc_layer(h, m0, m1, w, h_out, s_out):
    a = jnp.dot(h[...] + m0[...] + m1[...], w[...],
                preferred_element_type=_f32)
    ho, so = _silu_pair(a)
    h_out[...] = ho
    s_out[...] = so


def _tc_head(h1, m0, m1, w2, w2t, wr, ne, g2):
    a = jnp.dot(h1[...] + m0[...] + m1[...], w2[...],
                preferred_element_type=_f32)
    h2, s2 = _silu_pair(a)
    wrow = wr[...]
    ne[...] = jnp.sum(h2 * wrow, axis=1)[:, None]
    g2[...] = jnp.dot(s2 * wrow, w2t[...], preferred_element_type=_f32)


def _tc_energy(ne2d, out):
    out[...] = jnp.sum(ne2d[...], axis=1)


def _tc_bwd_dense(g2, q0, q1, s1, w1t, g1):
    g1[...] = jnp.dot((g2[...] + q0[...] + q1[...]) * s1[...], w1t[...],
                      preferred_element_type=_f32)


def _tc_gvec(mu2d, p1, p2, w1t, w2t, vec4, gv16):
    grbf = (jnp.dot(p1[...], w1t[...], preferred_element_type=_f32)
            + jnp.dot(p2[...], w2t[...], preferred_element_type=_f32))
    v = vec4[...]
    r = jnp.sqrt(jnp.sum(v * v, axis=1) + 1e-9)
    mu, rbf0, u, env = _rbf_env(r, mu2d[...])
    denv = jnp.where(u < 1.0,
                     -0.5 * jnp.pi * jnp.sin(jnp.pi * u) / RCUT, 0.0)
    drbf0 = -4.0 * (r[:, None] - mu[None, :]) * rbf0
    g_r = jnp.sum(grbf * (drbf0 * env[:, None] + rbf0 * denv[:, None]),
                  axis=1)
    gv = (g_r / r)[:, None] * v
    gv16[...] = jnp.concatenate(
        [gv, jnp.zeros((gv.shape[0], 12), _f32)], axis=1)


def _full(shape):
    zeros = (0,) * len(shape)
    return pl.BlockSpec(shape, lambda *_: zeros)


def _erow(width):
    return pl.BlockSpec((BE, width), lambda i: (i, 0))


def _nrow(width):
    return pl.BlockSpec((BN, width), lambda i: (i, 0))


def kernel(node_attrs, positions, edge_index, batch, ptr, shifts, cell,
           W_embed, W_rbf1, W_rbf2, W1, W2, W_readout):
    f32 = _f32
    src = edge_index[0].astype(jnp.int32)
    dst = edge_index[1].astype(jnp.int32)
    pos16 = jnp.zeros((N, 16), f32).at[:, :3].set(positions)
    sh4 = jnp.zeros((E, 4), f32).at[:, :3].set(shifts)
    mu2d = jnp.asarray(_MU).reshape(1, NRBF)

    gs, gd = _sc_gather_pos(pos16, src, dst)

    vec4, ew1, ew2 = pl.pallas_call(
        _tc_geom,
        grid=(E // BE,),
        in_specs=[_full((1, NRBF)), _erow(16), _erow(16), _erow(4),
                  _full((NRBF, HID)), _full((NRBF, HID))],
        out_specs=[_erow(4), _erow(HID), _erow(HID)],
        out_shape=[jax.ShapeDtypeStruct((E, 4), f32),
                   jax.ShapeDtypeStruct((E, HID), f32),
                   jax.ShapeDtypeStruct((E, HID), f32)],
    )(mu2d, gs, gd, sh4, W_rbf1, W_rbf2)

    h0 = pl.pallas_call(
        _tc_embed,
        grid=(N // BN,),
        in_specs=[_nrow(node_attrs.shape[1]), _full(W_embed.shape)],
        out_specs=_nrow(HID),
        out_shape=jax.ShapeDtypeStruct((N, HID), f32),
    )(node_attrs, W_embed)

    m1p = _sc_msg(h0, ew1, src, dst)

    h1, s1 = pl.pallas_call(
        _tc_layer,
        grid=(N // BN,),
        in_specs=[_nrow(HID), _nrow(HID), _nrow(HID), _full((HID, HID))],
        out_specs=[_nrow(HID), _nrow(HID)],
        out_shape=[jax.ShapeDtypeStruct((N, HID), f32),
                   jax.ShapeDtypeStruct((N, HID), f32)],
    )(h0, m1p[:N], m1p[N:], W1)

    m2p = _sc_msg(h1, ew2, src, dst)

    wrow = W_readout.reshape(1, HID)
    node_e, G2 = pl.pallas_call(
        _tc_head,
        grid=(N // BN,),
        in_specs=[_nrow(HID), _nrow(HID), _nrow(HID),
                  _full((HID, HID)), _full((HID, HID)), _full((1, HID))],
        out_specs=[_nrow(1), _nrow(HID)],
        out_shape=[jax.ShapeDtypeStruct((N, 1), f32),
                   jax.ShapeDtypeStruct((N, HID), f32)],
    )(h1, m2p[:N], m2p[N:], W2, W2.T, wrow)

    energy = pl.pallas_call(
        _tc_energy,
        in_specs=[_full((NSYS, N // NSYS))],
        out_specs=_full((NSYS,)),
        out_shape=jax.ShapeDtypeStruct((NSYS,), f32),
    )(node_e.reshape(NSYS, N // NSYS))

    prod2, q = _sc_bwd2(G2, h1, ew2, src, dst)

    G1 = pl.pallas_call(
        _tc_bwd_dense,
        grid=(N // BN,),
        in_specs=[_nrow(HID), _nrow(HID), _nrow(HID), _nrow(HID),
                  _full((HID, HID))],
        out_specs=_nrow(HID),
        out_shape=jax.ShapeDtypeStruct((N, HID), f32),
    )(G2, q[:N], q[N:], s1, W1.T)

    prod1 = _sc_bwd1(G1, h0, src, dst)

    gv16 = pl.pallas_call(
        _tc_gvec,
        grid=(E // BE,),
        in_specs=[_full((1, NRBF)), _erow(HID), _erow(HID),
                  _full((HID, NRBF)), _full((HID, NRBF)), _erow(4)],
        out_specs=_erow(16),
        out_shape=jax.ShapeDtypeStruct((E, 16), f32),
    )(mu2d, prod1, prod2, W_rbf1.T, W_rbf2.T, vec4)

    fp = _sc_forces(gv16, src, dst)
    forces = -(fp[:N, :3] + fp[N:, :3])
    return energy, forces


# speedup vs baseline: 3.9466x; 1.6650x over previous
import functools

import numpy as np
import jax
import jax.numpy as jnp
from jax import lax
from jax.experimental import pallas as pl
from jax.experimental.pallas import tpu as pltpu
from jax.experimental.pallas import tpu_sc as plsc

N = 10000
E = 320000
HID = 128
NRBF = 8
NSYS = 8
RCUT = 5.0

NC = 2
NS = 16
NW = NC * NS
EPW = E // NW
C = 80
C4 = 40
NCH = EPW // C
RPT = N // NS
DR = 25

_f32 = jnp.float32
_mesh = plsc.VectorSubcoreMesh(
    core_axis_name="c", subcore_axis_name="s", num_cores=NC, num_subcores=NS)
_params = pltpu.CompilerParams(use_tc_tiling_on_sc=False)
_params_nl = pltpu.CompilerParams(use_tc_tiling_on_sc=False,
                                  needs_layout_passes=False)


def _wid_base():
    cid = lax.axis_index("c")
    sid = lax.axis_index("s")
    return cid, sid, (sid * NC + cid) * EPW


def _zero_db(db, nrows):
    zero16 = jnp.zeros((16,), _f32)

    def body(i, _):
        for j in range(HID // 16):
            db[i, pl.ds(j * 16, 16)] = zero16
        return 0
    lax.fori_loop(0, nrows, body, 0)


def _drain(acc, db, out_hbm, cid, sid):
    for k in range(RPT // DR):
        r0 = sid * RPT + k * DR
        pltpu.sync_copy(acc.at[pl.ds(r0, DR)], db)
        pltpu.sync_copy(db, out_hbm.at[pl.ds(cid * N + r0, DR)])


def _zero_acc(acc, db, sid):
    _zero_db(db, DR)
    for k in range(RPT // DR):
        r0 = sid * RPT + k * DR
        pltpu.sync_copy(db, acc.at[pl.ds(r0, DR)])


def _pipe(nch, issue_lin, wait_lin, issue_gath, wait_gath, work, wait_w):
    issue_lin(0, 0)
    issue_lin(1, 1)
    wait_lin(0)
    issue_gath(0, 0)

    def it(i, _):
        b = i % 2
        o = 1 - b

        @pl.when(jnp.logical_and(i + 1 < nch, i >= 1))
        def _dw():
            wait_w(o)

        @pl.when(i + 1 < nch)
        def _ig():
            wait_lin(o)
            issue_gath(i + 1, o)

        wait_gath(b)
        work(i, b)

        @pl.when(i + 2 < nch)
        def _il():
            issue_lin(i + 2, b)
        return 0
    lax.fori_loop(0, nch, it, 0)
    wait_w(0)
    wait_w(1)


def _mk_lin(base, sem, pairs, c=C):
    def issue(i, b):
        st = base + i * c
        for hbm, buf2 in pairs:
            pltpu.async_copy(hbm.at[pl.ds(st, c)], buf2.at[b], sem.at[b])

    def wait(b):
        for hbm, buf2 in pairs:
            pltpu.make_async_copy(hbm.at[pl.ds(0, c)], buf2.at[b],
                                  sem.at[b]).wait()
    return issue, wait


def _mk_gath(sem, trips):
    def issue(i, b):
        for tab, idx2, rows2 in trips:
            pltpu.async_copy(tab.at[idx2.at[b]], rows2.at[b], sem.at[b])

    def wait(b):
        for tab, idx2, rows2 in trips:
            pltpu.make_async_copy(tab.at[idx2.at[b]], rows2.at[b],
                                  sem.at[b]).wait()
    return issue, wait


def _noop(*_a):
    return None


@functools.partial(
    pl.kernel,
    out_type=(jax.ShapeDtypeStruct((3, E), _f32),
              jax.ShapeDtypeStruct((3, E), _f32)),
    mesh=_mesh,
    compiler_params=_params_nl,
    scratch_types=(pltpu.VMEM((2, C), jnp.int32),
                   pltpu.VMEM((2, C), jnp.int32),
                   pltpu.VMEM((2, C, 16), _f32),
                   pltpu.VMEM((2, C, 16), _f32),
                   pltpu.VMEM((2, 3, C), _f32),
                   pltpu.VMEM((2, 3, C), _f32),
                   pltpu.SemaphoreType.DMA((2,)),
                   pltpu.SemaphoreType.DMA((2,)),
                   pltpu.SemaphoreType.DMA((2,))),
)
def _sc_gather_pos(pos_hbm, src_hbm, dst_hbm, gs_hbm, gd_hbm,
                   idx_s, idx_d, bs, bd, ts, td, sem_l, sem_g, sem_w):
    _, _, base = _wid_base()
    issue_lin, wait_lin = _mk_lin(base, sem_l,
                                  [(src_hbm, idx_s), (dst_hbm, idx_d)])
    issue_g, wait_g = _mk_gath(sem_g, [(pos_hbm, idx_s, bs),
                                       (pos_hbm, idx_d, bd)])
    lane = lax.iota(jnp.int32, 16)
    msk = lane < 3

    def work(i, b):
        tsb = ts.at[b]
        tdb = td.at[b]

        @plsc.parallel_loop(0, C, 1, unroll=4)
        def _t(e):
            ecol = jnp.full((16,), e, jnp.int32)
            plsc.store_scatter(tsb, [lane, ecol], bs[b, e, :], mask=msk)
            plsc.store_scatter(tdb, [lane, ecol], bd[b, e, :], mask=msk)
        st = base + i * C
        pltpu.async_copy(ts.at[b],
                         gs_hbm.at[pl.ds(0, 3), pl.ds(st, C)], sem_w.at[b])
        pltpu.async_copy(td.at[b],
                         gd_hbm.at[pl.ds(0, 3), pl.ds(st, C)], sem_w.at[b])

    def wait_w(b):
        pltpu.make_async_copy(ts.at[b],
                              gs_hbm.at[pl.ds(0, 3), pl.ds(0, C)],
                              sem_w.at[b]).wait()
        pltpu.make_async_copy(td.at[b],
                              gd_hbm.at[pl.ds(0, 3), pl.ds(0, C)],
                              sem_w.at[b]).wait()

    _pipe(NCH, issue_lin, wait_lin, issue_g, wait_g, work, wait_w)


@functools.partial(
    pl.kernel,
    out_type=jax.ShapeDtypeStruct((NC * N, HID), _f32),
    mesh=_mesh,
    compiler_params=_params,
    scratch_types=(pltpu.VMEM((2, C), jnp.int32),
                   pltpu.VMEM((2, C), jnp.int32),
                   pltpu.VMEM((2, C, HID), _f32),
                   pltpu.VMEM((2, C, HID), _f32),
                   pltpu.VMEM((DR, HID), _f32),
                   pltpu.MemorySpace.VMEM_SHARED((N, HID), _f32),
                   pltpu.SemaphoreType.DMA((2,)),
                   pltpu.SemaphoreType.DMA((2,))),
)
def _sc_msg(h_hbm, ew_hbm, src_hbm, dst_hbm, out_hbm,
            idx_s, idx_d, rows, ewb, db, acc, sem_l, sem_g):
    cid, sid, base = _wid_base()
    _zero_acc(acc, db, sid)
    plsc.subcore_barrier()

    issue_lin, wait_lin = _mk_lin(
        base, sem_l,
        [(src_hbm, idx_s), (dst_hbm, idx_d), (ew_hbm, ewb)])
    issue_g, wait_g = _mk_gath(sem_g, [(h_hbm, idx_s, rows)])

    def work(i, b):
        rb = rows.at[b]
        eb = ewb.at[b]

        @plsc.parallel_loop(0, C, 1, unroll=4)
        def _m(e):
            for j in range(HID // 16):
                sl = pl.ds(j * 16, 16)
                rb[e, sl] = rb[e, sl] * eb[e, sl]
        pltpu.sync_copy(rows.at[b], acc.at[idx_d.at[b]], add=True)

    _pipe(NCH, issue_lin, wait_lin, issue_g, wait_g, work, _noop)
    plsc.subcore_barrier()
    _drain(acc, db, out_hbm, cid, sid)


@functools.partial(
    pl.kernel,
    out_type=(jax.ShapeDtypeStruct((E, HID), _f32),
              jax.ShapeDtypeStruct((NC * N, HID), _f32)),
    mesh=_mesh,
    compiler_params=_params,
    scratch_types=(pltpu.VMEM((2, C4), jnp.int32),
                   pltpu.VMEM((2, C4), jnp.int32),
                   pltpu.VMEM((2, C4, HID), _f32),
                   pltpu.VMEM((2, C4, HID), _f32),
                   pltpu.VMEM((2, C4, HID), _f32),
                   pltpu.VMEM((DR, HID), _f32),
                   pltpu.MemorySpace.VMEM_SHARED((N, HID), _f32),
                   pltpu.SemaphoreType.DMA((2,)),
                   pltpu.SemaphoreType.DMA((2,)),
                   pltpu.SemaphoreType.DMA((2,))),
)
def _sc_bwd2(g2_hbm, h1_hbm, ew_hbm, src_hbm, dst_hbm, prod_hbm, out_hbm,
             idx_s, idx_d, g2r, h1r, ewb, db, acc, sem_l, sem_g, sem_w):
    cid, sid, base = _wid_base()
    _zero_acc(acc, db, sid)
    plsc.subcore_barrier()

    issue_lin, wait_lin = _mk_lin(
        base, sem_l,
        [(src_hbm, idx_s), (dst_hbm, idx_d), (ew_hbm, ewb)], c=C4)
    issue_g, wait_g = _mk_gath(sem_g, [(g2_hbm, idx_d, g2r),
                                       (h1_hbm, idx_s, h1r)])

    def work(i, b):
        gb = g2r.at[b]
        hb = h1r.at[b]
        eb = ewb.at[b]

        @plsc.parallel_loop(0, C4, 1, unroll=4)
        def _m(e):
            for j in range(HID // 16):
                sl = pl.ds(j * 16, 16)
                g = gb[e, sl]
                eb[e, sl] = g * eb[e, sl]
                hb[e, sl] = g * hb[e, sl]
        st = base + i * C4
        pltpu.async_copy(h1r.at[b], prod_hbm.at[pl.ds(st, C4)], sem_w.at[b])
        pltpu.sync_copy(ewb.at[b], acc.at[idx_s.at[b]], add=True)

    def wait_w(b):
        pltpu.make_async_copy(h1r.at[b], prod_hbm.at[pl.ds(0, C4)],
                              sem_w.at[b]).wait()

    _pipe(EPW // C4, issue_lin, wait_lin, issue_g, wait_g, work, wait_w)
    plsc.subcore_barrier()
    _drain(acc, db, out_hbm, cid, sid)


@functools.partial(
    pl.kernel,
    out_type=jax.ShapeDtypeStruct((E, HID), _f32),
    mesh=_mesh,
    compiler_params=_params,
    scratch_types=(pltpu.VMEM((2, C), jnp.int32),
                   pltpu.VMEM((2, C), jnp.int32),
                   pltpu.VMEM((2, C, HID), _f32),
                   pltpu.VMEM((2, C, HID), _f32),
                   pltpu.SemaphoreType.DMA((2,)),
                   pltpu.SemaphoreType.DMA((2,)),
                   pltpu.SemaphoreType.DMA((2,))),
)
def _sc_bwd1(g1_hbm, h0_hbm, src_hbm, dst_hbm, prod_hbm,
             idx_s, idx_d, g1r, h0r, sem_l, sem_g, sem_w):
    _, _, base = _wid_base()
    issue_lin, wait_lin = _mk_lin(base, sem_l,
                                  [(src_hbm, idx_s), (dst_hbm, idx_d)])
    issue_g, wait_g = _mk_gath(sem_g, [(g1_hbm, idx_d, g1r),
                                       (h0_hbm, idx_s, h0r)])

    def work(i, b):
        gb = g1r.at[b]
        hb = h0r.at[b]

        @plsc.parallel_loop(0, C, 1, unroll=4)
        def _m(e):
            for j in range(HID // 16):
                sl = pl.ds(j * 16, 16)
                hb[e, sl] = gb[e, sl] * hb[e, sl]
        st = base + i * C
        pltpu.async_copy(h0r.at[b], prod_hbm.at[pl.ds(st, C)], sem_w.at[b])

    def wait_w(b):
        pltpu.make_async_copy(h0r.at[b], prod_hbm.at[pl.ds(0, C)],
                              sem_w.at[b]).wait()

    _pipe(NCH, issue_lin, wait_lin, issue_g, wait_g, work, wait_w)


@functools.partial(
    pl.kernel,
    out_type=jax.ShapeDtypeStruct((NC * N, 16), _f32),
    mesh=_mesh,
    compiler_params=_params,
    scratch_types=(pltpu.VMEM((2, C), jnp.int32),
                   pltpu.VMEM((2, C), jnp.int32),
                   pltpu.VMEM((2, C, 16), _f32),
                   pltpu.VMEM((2, C, 16), _f32),
                   pltpu.VMEM((RPT, 16), _f32),
                   pltpu.MemorySpace.VMEM_SHARED((N, 16), _f32),
                   pltpu.SemaphoreType.DMA((2,))),
)
def _sc_forces(gv_hbm, src_hbm, dst_hbm, out_hbm,
               idx_s, idx_d, gvb, ngb, db, acc, sem_l):
    cid, sid, base = _wid_base()
    zero16 = jnp.zeros((16,), _f32)

    def zbody(i, _):
        db[i, :] = zero16
        return 0
    lax.fori_loop(0, RPT, zbody, 0)
    pltpu.sync_copy(db, acc.at[pl.ds(sid * RPT, RPT)])
    plsc.subcore_barrier()

    issue_lin, wait_lin = _mk_lin(
        base, sem_l,
        [(src_hbm, idx_s), (dst_hbm, idx_d), (gv_hbm, gvb)])

    def work(i, b):
        gb = gvb.at[b]
        nb = ngb.at[b]

        @plsc.parallel_loop(0, C, 1, unroll=4)
        def _m(e):
            nb[e, :] = -gb[e, :]
        pltpu.sync_copy(gvb.at[b], acc.at[idx_d.at[b]], add=True)
        pltpu.sync_copy(ngb.at[b], acc.at[idx_s.at[b]], add=True)

    _pipe(NCH, issue_lin, wait_lin, _noop, _noop, work, _noop)
    plsc.subcore_barrier()
    pltpu.sync_copy(acc.at[pl.ds(sid * RPT, RPT)], db)
    pltpu.sync_copy(db, out_hbm.at[pl.ds(cid * N + sid * RPT, RPT)])


BE = 2000
BEL = 2560
BN = 2000

_MU = np.linspace(0.0, RCUT, NRBF, dtype=np.float32)


def _tc_geom(mu2d, gs, gd, sh, wr1, wr2, vecT, ew1, ew2):
    v = gd[...] - gs[...] + sh[...]
    r = jnp.sqrt(jnp.sum(v * v, axis=0, keepdims=True) + 1e-9)
    t = r - mu2d[...]
    rbf0 = jnp.exp(-2.0 * t * t)
    u = r / RCUT
    env = jnp.where(u < 1.0, 0.5 * (jnp.cos(jnp.pi * u) + 1.0), 0.0)
    rbf = rbf0 * env
    vecT[...] = v
    dn = (((0,), (0,)), ((), ()))
    ew1[...] = lax.dot_general(rbf, wr1[...], dn,
                               preferred_element_type=_f32)
    ew2[...] = lax.dot_general(rbf, wr2[...], dn,
                               preferred_element_type=_f32)


def _tc_embed(attrs, we, h0):
    h0[...] = jnp.dot(attrs[...], we[...], preferred_element_type=_f32)


def _silu_pair(a):
    sg = 1.0 / (1.0 + jnp.exp(-a))
    return a * sg, sg * (1.0 + a * (1.0 - sg))


def _tc_layer(h, m0, m1, w, h_out, s_out):
    a = jnp.dot(h[...] + m0[...] + m1[...], w[...],
                preferred_element_type=_f32)
    ho, so = _silu_pair(a)
    h_out[...] = ho
    s_out[...] = so


def _tc_head(h1, m0, m1, w2, w2t, wr, ne, g2):
    a = jnp.dot(h1[...] + m0[...] + m1[...], w2[...],
                preferred_element_type=_f32)
    h2, s2 = _silu_pair(a)
    wrow = wr[...]
    ne[...] = jnp.sum(h2 * wrow, axis=1)[:, None]
    g2[...] = jnp.dot(s2 * wrow, w2t[...], preferred_element_type=_f32)


def _tc_energy(ne2d, out):
    out[...] = jnp.sum(ne2d[...], axis=1)


def _tc_bwd_dense(g2, q0, q1, s1, w1t, g1):
    g1[...] = jnp.dot((g2[...] + q0[...] + q1[...]) * s1[...], w1t[...],
                      preferred_element_type=_f32)


def _tc_gvec(mu2d, p1, p2, w1t, w2t, vecT, gv16):
    gr8 = (jnp.dot(p1[...], w1t[...], preferred_element_type=_f32)
           + jnp.dot(p2[...], w2t[...], preferred_element_type=_f32))
    gr = jnp.transpose(gr8)
    v = vecT[...]
    r = jnp.sqrt(jnp.sum(v * v, axis=0, keepdims=True) + 1e-9)
    t = r - mu2d[...]
    rbf0 = jnp.exp(-2.0 * t * t)
    u = r / RCUT
    env = jnp.where(u < 1.0, 0.5 * (jnp.cos(jnp.pi * u) + 1.0), 0.0)
    denv = jnp.where(u < 1.0,
                     -0.5 * jnp.pi * jnp.sin(jnp.pi * u) / RCUT, 0.0)
    drbf0 = -4.0 * t * rbf0
    g_r = jnp.sum(gr * (drbf0 * env + rbf0 * denv), axis=0,
                  keepdims=True)
    gv = jnp.transpose((g_r / r) * v)
    gv16[...] = jnp.concatenate(
        [gv, jnp.zeros((gv.shape[0], 13), _f32)], axis=1)


def _full(shape):
    zeros = (0,) * len(shape)
    return pl.BlockSpec(shape, lambda *_: zeros)


def _erow(width):
    return pl.BlockSpec((BE, width), lambda i: (i, 0))


def _elrow(width):
    return pl.BlockSpec((BEL, width), lambda i: (i, 0))


def _ecol(height):
    return pl.BlockSpec((height, BEL), lambda i: (0, i))


def _nrow(width):
    return pl.BlockSpec((BN, width), lambda i: (i, 0))


def kernel(node_attrs, positions, edge_index, batch, ptr, shifts, cell,
           W_embed, W_rbf1, W_rbf2, W1, W2, W_readout):
    f32 = _f32
    src = edge_index[0].astype(jnp.int32)
    dst = edge_index[1].astype(jnp.int32)
    pos16 = jnp.zeros((N, 16), f32).at[:, :3].set(positions)
    shT = shifts.T
    mu2d = jnp.asarray(_MU).reshape(NRBF, 1)

    gs, gd = _sc_gather_pos(pos16, src, dst)

    vecT, ew1, ew2 = pl.pallas_call(
        _tc_geom,
        grid=(E // BEL,),
        in_specs=[_full((NRBF, 1)), _ecol(3), _ecol(3), _ecol(3),
                  _full((NRBF, HID)), _full((NRBF, HID))],
        out_specs=[_ecol(3), _elrow(HID), _elrow(HID)],
        out_shape=[jax.ShapeDtypeStruct((3, E), f32),
                   jax.ShapeDtypeStruct((E, HID), f32),
                   jax.ShapeDtypeStruct((E, HID), f32)],
    )(mu2d, gs, gd, shT, W_rbf1, W_rbf2)

    h0 = pl.pallas_call(
        _tc_embed,
        grid=(N // BN,),
        in_specs=[_nrow(node_attrs.shape[1]), _full(W_embed.shape)],
        out_specs=_nrow(HID),
        out_shape=jax.ShapeDtypeStruct((N, HID), f32),
    )(node_attrs, W_embed)

    m1p = _sc_msg(h0, ew1, src, dst)

    h1, s1 = pl.pallas_call(
        _tc_layer,
        grid=(N // BN,),
        in_specs=[_nrow(HID), _nrow(HID), _nrow(HID), _full((HID, HID))],
        out_specs=[_nrow(HID), _nrow(HID)],
        out_shape=[jax.ShapeDtypeStruct((N, HID), f32),
                   jax.ShapeDtypeStruct((N, HID), f32)],
    )(h0, m1p[:N], m1p[N:], W1)

    m2p = _sc_msg(h1, ew2, src, dst)

    wrow = W_readout.reshape(1, HID)
    node_e, G2 = pl.pallas_call(
        _tc_head,
        grid=(N // BN,),
        in_specs=[_nrow(HID), _nrow(HID), _nrow(HID),
                  _full((HID, HID)), _full((HID, HID)), _full((1, HID))],
        out_specs=[_nrow(1), _nrow(HID)],
        out_shape=[jax.ShapeDtypeStruct((N, 1), f32),
                   jax.ShapeDtypeStruct((N, HID), f32)],
    )(h1, m2p[:N], m2p[N:], W2, W2.T, wrow)

    energy = pl.pallas_call(
        _tc_energy,
        in_specs=[_full((NSYS, N // NSYS))],
        out_specs=_full((NSYS,)),
        out_shape=jax.ShapeDtypeStruct((NSYS,), f32),
    )(node_e.reshape(NSYS, N // NSYS))

    prod2, q = _sc_bwd2(G2, h1, ew2, src, dst)

    G1 = pl.pallas_call(
        _tc_bwd_dense,
        grid=(N // BN,),
        in_specs=[_nrow(HID), _nrow(HID), _nrow(HID), _nrow(HID),
                  _full((HID, HID))],
        out_specs=_nrow(HID),
        out_shape=jax.ShapeDtypeStruct((N, HID), f32),
    )(G2, q[:N], q[N:], s1, W1.T)

    prod1 = _sc_bwd1(G1, h0, src, dst)

    gv16 = pl.pallas_call(
        _tc_gvec,
        grid=(E // BEL,),
        in_specs=[_full((NRBF, 1)), _elrow(HID), _elrow(HID),
                  _full((HID, NRBF)), _full((HID, NRBF)), _ecol(3)],
        out_specs=_elrow(16),
        out_shape=jax.ShapeDtypeStruct((E, 16), f32),
    )(mu2d, prod1, prod2, W_rbf1.T, W_rbf2.T, vecT)

    fp = _sc_forces(gv16, src, dst)
    forces = -(fp[:N, :3] + fp[N:, :3])
    return energy, forces
